# baseline (speedup 1.0000x reference)
import functools

import jax
import jax.numpy as jnp
from jax import lax
from jax.experimental import pallas as pl
from jax.experimental.pallas import tpu as pltpu
from jax.experimental.pallas import tpu_sc as plsc

D = 32
NC, NS = 2, 16
NW = NC * NS
CB = 128
FP = CB + 1


def _body(n_chunks_per_w, chunks_per_s, n_ann, n_chunks_total,
          ann_rm, qtab, ai, qi, px5, feat5,
          idxa0, idxa1, idxq0, idxq1, qtab_v, rows0, rows1,
          feat0, feat1, xv0, xv1,
          sem_g0, sem_g1, sem_x0, sem_x1, sem_w):
    wid = lax.axis_index("s") * NC + lax.axis_index("c")

    pltpu.sync_copy(qtab, qtab_v)
    f16a = lax.iota(jnp.int32, 16)
    f16b = f16a + 16

    ti16a, f016a = f16a // 8, f16a % 8
    ti16b, f016b = f16b // 8, f16b % 8
    bufs = (
        (idxa0, idxq0, rows0, feat0, xv0, sem_g0, sem_x0),
        (idxa1, idxq1, rows1, feat1, xv1, sem_g1, sem_x1),
    )
    base = wid * n_chunks_per_w
    GK = 10

    def fire(g, k):
        idxa, idxq, rows, _, xv, sem_g, sem_x = bufs[k % 2]
        c = base + g * GK + k
        s = c // chunks_per_s
        tj = c % chunks_per_s
        pltpu.sync_copy(ai.at[pl.ds(c * CB, CB)], idxa)
        pltpu.sync_copy(qi.at[pl.ds(c * CB, CB)], idxq.at[pl.ds(0, CB)])

        def remap(r, c2):
            v = idxa[pl.ds(r * 16, 16)]
            idxa[pl.ds(r * 16, 16)] = jnp.where(v < 0, n_ann, v)
            return c2
        lax.fori_loop(0, CB // 16, remap, 0)
        hg = pltpu.async_copy(ann_rm.at[idxa], rows, sem_g)
        hx = pltpu.async_copy(px5.at[s, :, tj], xv, sem_x)
        return hg, hx

    def process(g, k, handles):
        idxa, idxq, rows, feat, xv, sem_g, sem_x = bufs[k % 2]
        c = base + g * GK + k
        s = c // chunks_per_s
        tj = c % chunks_per_s
        hg, hx = handles
        hg.wait()
        nxt = fire(g, k + 1) if k + 1 < GK else None

        def row4(b4, c2):
            for u in range(4):
                b = b4 * 4 + u
                qb = idxq[pl.ds(b, 16)][0] * D
                v0 = rows[b, pl.ds(0, 16)] + qtab_v[pl.ds(qb, 16)]
                v1 = rows[b, pl.ds(16, 16)] + qtab_v[pl.ds(qb + 16, 16)]
                bcol = jnp.full((16,), b, jnp.int32)
                plsc.store_scatter(feat, [ti16a, f016a, bcol], v0)
                plsc.store_scatter(feat, [ti16b, f016b, bcol], v1)
            return c2
        lax.fori_loop(0, CB // 4, row4, 0)

        hx.wait()
        ce = pltpu.async_copy(feat.at[:, :, pl.ds(0, CB)],
                              feat5.at[s, pl.ds(0, D // 8), tj], sem_w)
        cx = pltpu.async_copy(xv, feat5.at[s, pl.ds(D // 8, D // 8), tj],
                              sem_w)
        ce.wait()
        cx.wait()
        return nxt

    def group(g, carry):
        handles = fire(g, 0)
        for k in range(GK):
            handles = process(g, k, handles)
        return carry

    lax.fori_loop(0, n_chunks_per_w // GK, group, 0)


def kernel(x, annotators, questions, annotator_embedding, question_embedding):
    B, S, XF = x.shape
    N = B * S
    n_ann = annotator_embedding.shape[0] - 1
    assert B % CB == 0 and N % (NW * CB) == 0
    chunks_per_s = B // CB
    n_chunks_per_w = N // (NW * CB)

    ai = annotators.T.reshape(N).astype(jnp.int32)
    qi = questions.T.reshape(N).astype(jnp.int32)
    qtab = question_embedding.reshape(-1)

    param_x = x[:, :, 1:]
    px5 = (param_x.transpose(1, 2, 0)
           .reshape(S, D // 8, 8, B // CB, CB)
           .transpose(0, 1, 3, 2, 4))

    mesh = plsc.VectorSubcoreMesh(core_axis_name="c", subcore_axis_name="s")
    feat5 = pl.kernel(
        functools.partial(_body, n_chunks_per_w, chunks_per_s, n_ann,
                          N // CB),
        out_type=jax.ShapeDtypeStruct((S, 2 * D // 8, B // CB, 8, CB),
                                      jnp.float32),
        mesh=mesh,
        compiler_params=pltpu.CompilerParams(
            use_tc_tiling_on_sc=False, needs_layout_passes=False),
        scratch_types=[
            pltpu.VMEM((CB,), jnp.int32),
            pltpu.VMEM((CB,), jnp.int32),
            pltpu.VMEM((CB + 16,), jnp.int32),
            pltpu.VMEM((CB + 16,), jnp.int32),
            pltpu.VMEM((question_embedding.size,), jnp.float32),
            pltpu.VMEM((CB, D), jnp.float32),
            pltpu.VMEM((CB, D), jnp.float32),
            pltpu.VMEM((D // 8, 8, FP), jnp.float32),
            pltpu.VMEM((D // 8, 8, FP), jnp.float32),
            pltpu.VMEM((D // 8, 8, CB), jnp.float32),
            pltpu.VMEM((D // 8, 8, CB), jnp.float32),
            pltpu.SemaphoreType.DMA,
            pltpu.SemaphoreType.DMA,
            pltpu.SemaphoreType.DMA,
            pltpu.SemaphoreType.DMA,
            pltpu.SemaphoreType.DMA,
        ],
    )(annotator_embedding, qtab, ai, qi, px5)

    feature_x = feat5.transpose(2, 4, 0, 1, 3).reshape(B, S, 2 * D)
    return feature_x, param_x


# speedup vs baseline: 1.0016x; 1.0016x over previous
import functools

import jax
import jax.numpy as jnp
from jax import lax
from jax.experimental import pallas as pl
from jax.experimental.pallas import tpu as pltpu
from jax.experimental.pallas import tpu_sc as plsc

D = 32
NC, NS = 2, 16
NW = NC * NS
CB = 128
FP = CB + 1


def _body(n_chunks_per_w, chunks_per_s, n_ann, n_chunks_total,
          ann_rm, qtab, ai, qi, px5, feat5,
          idxa0, idxa1, idxq0, idxq1, qtab_v, rows0, rows1,
          feat0, feat1, xv0, xv1,
          sem_g0, sem_g1, sem_x0, sem_x1, sem_w):
    wid = lax.axis_index("s") * NC + lax.axis_index("c")

    pltpu.sync_copy(qtab, qtab_v)
    f16a = lax.iota(jnp.int32, 16)
    f16b = f16a + 16

    ti16a, f016a = f16a // 8, f16a % 8
    ti16b, f016b = f16b // 8, f16b % 8
    bufs = (
        (idxa0, idxq0, rows0, feat0, xv0, sem_g0, sem_x0),
        (idxa1, idxq1, rows1, feat1, xv1, sem_g1, sem_x1),
    )
    base = wid * n_chunks_per_w
    GK = 25

    def fire(g, k):
        idxa, idxq, rows, _, xv, sem_g, sem_x = bufs[k % 2]
        c = base + g * GK + k
        s = c // chunks_per_s
        tj = c % chunks_per_s
        pltpu.sync_copy(ai.at[pl.ds(c * CB, CB)], idxa)
        pltpu.sync_copy(qi.at[pl.ds(c * CB, CB)], idxq.at[pl.ds(0, CB)])

        def remap(r, c2):
            v = idxa[pl.ds(r * 16, 16)]
            idxa[pl.ds(r * 16, 16)] = jnp.where(v < 0, n_ann, v)
            return c2
        lax.fori_loop(0, CB // 16, remap, 0)
        hg = pltpu.async_copy(ann_rm.at[idxa], rows, sem_g)
        hx = pltpu.async_copy(px5.at[s, :, tj], xv, sem_x)
        return hg, hx

    def process(g, k, handles):
        idxa, idxq, rows, feat, xv, sem_g, sem_x = bufs[k % 2]
        c = base + g * GK + k
        s = c // chunks_per_s
        tj = c % chunks_per_s
        hg, hx = handles
        nxt = fire(g, k + 1) if k + 1 < GK else None
        hg.wait()

        def row4(b4, c2):
            for u in range(4):
                b = b4 * 4 + u
                qb = idxq[pl.ds(b, 16)][0] * D
                v0 = rows[b, pl.ds(0, 16)] + qtab_v[pl.ds(qb, 16)]
                v1 = rows[b, pl.ds(16, 16)] + qtab_v[pl.ds(qb + 16, 16)]
                bcol = jnp.full((16,), b, jnp.int32)
                plsc.store_scatter(feat, [ti16a, f016a, bcol], v0)
                plsc.store_scatter(feat, [ti16b, f016b, bcol], v1)
            return c2
        lax.fori_loop(0, CB // 4, row4, 0)

        hx.wait()
        ce = pltpu.async_copy(feat.at[:, :, pl.ds(0, CB)],
                              feat5.at[s, pl.ds(0, D // 8), tj], sem_w)
        cx = pltpu.async_copy(xv, feat5.at[s, pl.ds(D // 8, D // 8), tj],
                              sem_w)
        ce.wait()
        cx.wait()
        return nxt

    def group(g, carry):
        handles = fire(g, 0)
        for k in range(GK):
            handles = process(g, k, handles)
        return carry

    lax.fori_loop(0, n_chunks_per_w // GK, group, 0)


def kernel(x, annotators, questions, annotator_embedding, question_embedding):
    B, S, XF = x.shape
    N = B * S
    n_ann = annotator_embedding.shape[0] - 1
    assert B % CB == 0 and N % (NW * CB) == 0
    chunks_per_s = B // CB
    n_chunks_per_w = N // (NW * CB)

    ai = annotators.T.reshape(N).astype(jnp.int32)
    qi = questions.T.reshape(N).astype(jnp.int32)
    qtab = question_embedding.reshape(-1)

    param_x = x[:, :, 1:]
    px5 = (param_x.transpose(1, 2, 0)
           .reshape(S, D // 8, 8, B // CB, CB)
           .transpose(0, 1, 3, 2, 4))

    mesh = plsc.VectorSubcoreMesh(core_axis_name="c", subcore_axis_name="s")
    feat5 = pl.kernel(
        functools.partial(_body, n_chunks_per_w, chunks_per_s, n_ann,
                          N // CB),
        out_type=jax.ShapeDtypeStruct((S, 2 * D // 8, B // CB, 8, CB),
                                      jnp.float32),
        mesh=mesh,
        compiler_params=pltpu.CompilerParams(
            use_tc_tiling_on_sc=False, needs_layout_passes=False),
        scratch_types=[
            pltpu.VMEM((CB,), jnp.int32),
            pltpu.VMEM((CB,), jnp.int32),
            pltpu.VMEM((CB + 16,), jnp.int32),
            pltpu.VMEM((CB + 16,), jnp.int32),
            pltpu.VMEM((question_embedding.size,), jnp.float32),
            pltpu.VMEM((CB, D), jnp.float32),
            pltpu.VMEM((CB, D), jnp.float32),
            pltpu.VMEM((D // 8, 8, FP), jnp.float32),
            pltpu.VMEM((D // 8, 8, FP), jnp.float32),
            pltpu.VMEM((D // 8, 8, CB), jnp.float32),
            pltpu.VMEM((D // 8, 8, CB), jnp.float32),
            pltpu.SemaphoreType.DMA,
            pltpu.SemaphoreType.DMA,
            pltpu.SemaphoreType.DMA,
            pltpu.SemaphoreType.DMA,
            pltpu.SemaphoreType.DMA,
        ],
    )(annotator_embedding, qtab, ai, qi, px5)

    feature_x = feat5.transpose(2, 4, 0, 1, 3).reshape(B, S, 2 * D)
    return feature_x, param_x


# speedup vs baseline: 1.0185x; 1.0168x over previous
import functools

import jax
import jax.numpy as jnp
from jax import lax
from jax.experimental import pallas as pl
from jax.experimental.pallas import tpu as pltpu
from jax.experimental.pallas import tpu_sc as plsc

D = 32
NC, NS = 2, 16
NW = NC * NS
CB = 128
FP = CB + 1
NB = 3


def _body(n_chunks_per_w, chunks_per_s, n_ann, n_chunks_total,
          ann_rm, qtab, ai, qi, px5, feat5,
          idxa0, idxa1, idxa2, idxq0, idxq1, idxq2, qtab_v,
          rows0, rows1, rows2, feat0, feat1, feat2, xv0, xv1, xv2,
          sem_g0, sem_g1, sem_g2, sem_x0, sem_x1, sem_x2,
          sem_w0, sem_w1, sem_w2):
    wid = lax.axis_index("s") * NC + lax.axis_index("c")

    pltpu.sync_copy(qtab, qtab_v)
    f16a = lax.iota(jnp.int32, 16)
    f16b = f16a + 16

    ti16a, f016a = f16a // 8, f16a % 8
    ti16b, f016b = f16b // 8, f16b % 8
    bufs = (
        (idxa0, idxq0, rows0, feat0, xv0, sem_g0, sem_x0, sem_w0),
        (idxa1, idxq1, rows1, feat1, xv1, sem_g1, sem_x1, sem_w1),
        (idxa2, idxq2, rows2, feat2, xv2, sem_g2, sem_x2, sem_w2),
    )
    base = wid * n_chunks_per_w
    GK = 25

    def fire(g, k):
        idxa, idxq, rows, _, xv, sem_g, sem_x, _ = bufs[k % NB]
        c = base + g * GK + k
        s = c // chunks_per_s
        tj = c % chunks_per_s
        pltpu.sync_copy(ai.at[pl.ds(c * CB, CB)], idxa)
        pltpu.sync_copy(qi.at[pl.ds(c * CB, CB)], idxq.at[pl.ds(0, CB)])

        def remap(r, c2):
            v = idxa[pl.ds(r * 16, 16)]
            idxa[pl.ds(r * 16, 16)] = jnp.where(v < 0, n_ann, v)
            return c2
        lax.fori_loop(0, CB // 16, remap, 0)
        hg = pltpu.async_copy(ann_rm.at[idxa], rows, sem_g)
        hx = pltpu.async_copy(px5.at[s, :, tj], xv, sem_x)
        return hg, hx

    def drain(outh, b):
        if outh[b] is not None:
            outh[b][0].wait()
            outh[b][1].wait()
            outh[b] = None

    def process(g, k, handles, outh):
        idxa, idxq, rows, feat, xv, sem_g, sem_x, sem_w = bufs[k % NB]
        c = base + g * GK + k
        s = c // chunks_per_s
        tj = c % chunks_per_s
        hg, hx = handles
        if k + 1 < GK:
            drain(outh, (k + 1) % NB)
            nxt = fire(g, k + 1)
        else:
            nxt = None
        drain(outh, k % NB)
        hg.wait()

        def row4(b4, c2):
            for u in range(4):
                b = b4 * 4 + u
                qb = idxq[pl.ds(b, 16)][0] * D
                v0 = rows[b, pl.ds(0, 16)] + qtab_v[pl.ds(qb, 16)]
                v1 = rows[b, pl.ds(16, 16)] + qtab_v[pl.ds(qb + 16, 16)]
                bcol = jnp.full((16,), b, jnp.int32)
                plsc.store_scatter(feat, [ti16a, f016a, bcol], v0)
                plsc.store_scatter(feat, [ti16b, f016b, bcol], v1)
            return c2
        lax.fori_loop(0, CB // 4, row4, 0)

        hx.wait()
        ce = pltpu.async_copy(feat.at[:, :, pl.ds(0, CB)],
                              feat5.at[s, pl.ds(0, D // 8), tj], sem_w)
        cx = pltpu.async_copy(xv, feat5.at[s, pl.ds(D // 8, D // 8), tj],
                              sem_w)
        outh[k % NB] = (ce, cx)
        return nxt

    def group(g, carry):
        outh = [None] * NB
        handles = fire(g, 0)
        for k in range(GK):
            handles = process(g, k, handles, outh)
        for b in range(NB):
            drain(outh, b)
        return carry

    lax.fori_loop(0, n_chunks_per_w // GK, group, 0)


def kernel(x, annotators, questions, annotator_embedding, question_embedding):
    B, S, XF = x.shape
    N = B * S
    n_ann = annotator_embedding.shape[0] - 1
    assert B % CB == 0 and N % (NW * CB) == 0
    chunks_per_s = B // CB
    n_chunks_per_w = N // (NW * CB)

    ai = annotators.T.reshape(N).astype(jnp.int32)
    qi = questions.T.reshape(N).astype(jnp.int32)
    qtab = question_embedding.reshape(-1)

    param_x = x[:, :, 1:]
    px5 = (param_x.transpose(1, 2, 0)
           .reshape(S, D // 8, 8, B // CB, CB)
           .transpose(0, 1, 3, 2, 4))

    mesh = plsc.VectorSubcoreMesh(core_axis_name="c", subcore_axis_name="s")
    feat5 = pl.kernel(
        functools.partial(_body, n_chunks_per_w, chunks_per_s, n_ann,
                          N // CB),
        out_type=jax.ShapeDtypeStruct((S, 2 * D // 8, B // CB, 8, CB),
                                      jnp.float32),
        mesh=mesh,
        compiler_params=pltpu.CompilerParams(
            use_tc_tiling_on_sc=False, needs_layout_passes=False),
        scratch_types=[
            pltpu.VMEM((CB,), jnp.int32),
            pltpu.VMEM((CB,), jnp.int32),
            pltpu.VMEM((CB,), jnp.int32),
            pltpu.VMEM((CB + 16,), jnp.int32),
            pltpu.VMEM((CB + 16,), jnp.int32),
            pltpu.VMEM((CB + 16,), jnp.int32),
            pltpu.VMEM((question_embedding.size,), jnp.float32),
            pltpu.VMEM((CB, D), jnp.float32),
            pltpu.VMEM((CB, D), jnp.float32),
            pltpu.VMEM((CB, D), jnp.float32),
            pltpu.VMEM((D // 8, 8, FP), jnp.float32),
            pltpu.VMEM((D // 8, 8, FP), jnp.float32),
            pltpu.VMEM((D // 8, 8, FP), jnp.float32),
            pltpu.VMEM((D // 8, 8, CB), jnp.float32),
            pltpu.VMEM((D // 8, 8, CB), jnp.float32),
            pltpu.VMEM((D // 8, 8, CB), jnp.float32),
            pltpu.SemaphoreType.DMA,
            pltpu.SemaphoreType.DMA,
            pltpu.SemaphoreType.DMA,
            pltpu.SemaphoreType.DMA,
            pltpu.SemaphoreType.DMA,
            pltpu.SemaphoreType.DMA,
            pltpu.SemaphoreType.DMA,
            pltpu.SemaphoreType.DMA,
            pltpu.SemaphoreType.DMA,
        ],
    )(annotator_embedding, qtab, ai, qi, px5)

    feature_x = feat5.transpose(2, 4, 0, 1, 3).reshape(B, S, 2 * D)
    return feature_x, param_x
